# baseline (speedup 1.0000x reference)
import functools
import math

import numpy as np
import jax
import jax.numpy as jnp
from jax import lax
from jax.experimental import pallas as pl
from jax.experimental.pallas import tpu as pltpu
from jax.experimental.pallas import tpu_sc as plsc

NC = 2
NS = 16
NW = NC * NS

_PERM_NP = np.zeros((96, 96), np.float32)
for _u in range(32):
    for _i in range(3):
        _PERM_NP[3 * _u + _i, 32 * _i + _u] = 1.0

_LOG2 = math.log(2.0)
_C_IP = 1.0 / (3.0 * math.sqrt(3.0))
_C0 = math.sqrt(0.5)
_C1 = math.sqrt(1.5)


def _ssp(v):
    return jnp.maximum(v, 0.0) + jnp.log1p(jnp.exp(-jnp.abs(v))) - _LOG2


def _dot(a, b):
    return jax.lax.dot_general(a, b, (((1,), (0,)), ((), ())),
                               preferred_element_type=jnp.float32,
                               precision=jax.lax.Precision.HIGHEST)


def _node_body(x_ref, perm_ref, wpre0_ref, bpre0_ref, mpre1_ref, wp_ref,
               wg1_ref, bg1_ref, wg2_ref, bg2_ref, wnode0_ref, bnode0_ref,
               mnode1_ref, t1_ref, t2_ref):
    x = x_ref[...]
    x0 = x[:, :32]
    x1u = x[:, 32:]
    x1im = _dot(x1u, perm_ref[...])
    pre0 = _dot(x0, wpre0_ref[...]) + bpre0_ref[...]
    pre1im = _dot(x1u, mpre1_ref[...])
    p = _dot(pre0, wp_ref[...])
    n1 = jnp.sqrt(x1im[:, :32] ** 2 + x1im[:, 32:64] ** 2
                  + x1im[:, 64:96] ** 2 + 1e-12)
    f0 = jnp.concatenate([x0, n1], axis=1)
    h = _dot(f0, wg1_ref[...]) + bg1_ref[...]
    g = _dot(h * jax.nn.sigmoid(h), wg2_ref[...]) + bg2_ref[...]
    g1 = g[:, 32:64]
    g1t = jnp.concatenate([g1, g1, g1], axis=1)
    xg1im = x1im * g1t
    xl0 = _dot(g[:, :32], wnode0_ref[...]) + bnode0_ref[...]
    xl1im = _dot(xg1im, mnode1_ref[...])
    t1_ref[...] = jnp.concatenate([p, pre1im], axis=1)
    t2_ref[...] = jnp.concatenate(
        [pre1im, xl0, xl1im, jnp.zeros_like(xl0)], axis=1)


def _edge_body(g1_ref, g2_ref, attr_ref, sh_ref, wa_ref, wb_ref, ef_ref):
    g1 = g1_ref[...]
    g2 = g2_ref[...]
    pd = g1[:, :32]
    pd1 = g1[:, 32:128]
    ps1 = g2[:, :96]
    x0s = g2[:, 96:128]
    x1s = g2[:, 128:224]
    prod = pd1 * ps1
    ip1 = (prod[:, :32] + prod[:, 32:64] + prod[:, 64:96]) * _C_IP
    u = jnp.concatenate([ip1, attr_ref[...]], axis=1)
    huv = _dot(u, wa_ref[...])
    hs = _ssp(huv[:, :32] + pd)
    hf = _ssp(huv[:, 32:64])
    z = jnp.concatenate([hs, hf], axis=1)
    wz = _dot(z, wb_ref[...])
    w = wz[:, :64] * wz[:, 64:128] * sh_ref[...]
    w1 = w[:, 32:64]
    ef_ref[:, :32] = x0s * w[:, :32]
    ef_ref[:, 32:128] = x1s * jnp.concatenate([w1, w1, w1], axis=1)


def _final_body(pa_ref, pb_ref, t2_ref, wo0_ref, bo0_ref, mout_ref, o_ref):
    pa = pa_ref[...]
    pb = pb_ref[...]
    t2 = t2_ref[...]
    p0 = pa[0] + pa[1]
    p1 = pb[0] + pb[1]
    acc0 = p0[:, :32] + p1[:, :32] + t2[:, 96:128]
    acc1 = p0[:, 32:] + p1[:, 32:] + t2[:, 128:224]
    o0 = _dot(acc0, wo0_ref[...]) + bo0_ref[...]
    o1 = _dot(acc1, mout_ref[...])
    o_ref[...] = jnp.concatenate([o0, o1], axis=1)


def _make_gather(n, es, ch, si):
    nch = (es // NW) // ch
    epw = es // NW
    mesh = plsc.VectorSubcoreMesh(core_axis_name="c", subcore_axis_name="s")

    @functools.partial(
        pl.kernel,
        out_type=[jax.ShapeDtypeStruct((es, 128), jnp.float32),
                  jax.ShapeDtypeStruct((es, 256), jnp.float32)],
        mesh=mesh,
        scratch_types=[
            pltpu.VMEM((nch, ch), jnp.int32),
            pltpu.VMEM((nch, ch), jnp.int32),
            pltpu.VMEM((ch, 128), jnp.float32),
            pltpu.VMEM((ch, 256), jnp.float32),
            pltpu.SemaphoreType.DMA,
            pltpu.SemaphoreType.DMA,
        ],
        compiler_params=pltpu.CompilerParams(use_tc_tiling_on_sc=False),
    )
    def gather(t1, t2, dsti, srci, g1, g2, idxd, idxs, b1, b2, sem1, sem2):
        c = lax.axis_index("c")
        s = lax.axis_index("s")
        wid = s * NC + c
        base = wid * epw
        off = si * es + wid * epw
        cps = []
        for j in range(nch):
            cps.append(pltpu.async_copy(
                dsti.at[pl.ds(off + j * ch, ch)], idxd.at[j], sem1))
            cps.append(pltpu.async_copy(
                srci.at[pl.ds(off + j * ch, ch)], idxs.at[j], sem2))
        for cp in cps:
            cp.wait()

        def body(j, carry):
            cp1 = pltpu.async_copy(t1.at[idxd.at[j]], b1, sem1)
            cp2 = pltpu.async_copy(t2.at[idxs.at[j]], b2, sem2)
            cp1.wait()
            cp2.wait()
            pltpu.sync_copy(b1, g1.at[pl.ds(base + j * ch, ch)])
            pltpu.sync_copy(b2, g2.at[pl.ds(base + j * ch, ch)])
            return carry

        lax.fori_loop(0, nch, body, 0)

    return gather


def _make_scatter(n_pad, es, ch, slabs):
    nch = (es // NW) // ch
    epw = es // NW
    npt = n_pad // NS
    nslab = len(slabs)
    mesh = plsc.VectorSubcoreMesh(core_axis_name="c", subcore_axis_name="s")

    @functools.partial(
        pl.kernel,
        out_type=jax.ShapeDtypeStruct((NC, n_pad, 128), jnp.float32),
        mesh=mesh,
        scratch_types=[
            pltpu.VMEM((nch, ch), jnp.int32),
            pltpu.VMEM((ch, 128), jnp.float32),
            pltpu.VMEM_SHARED((n_pad, 128), jnp.float32),
            pltpu.SemaphoreType.DMA,
        ],
        compiler_params=pltpu.CompilerParams(use_tc_tiling_on_sc=False),
    )
    def scatter(*refs):
        efs = refs[:nslab]
        dsti = refs[nslab]
        zeros = refs[nslab + 1]
        out = refs[nslab + 2]
        idxd, buf, shared, isem = refs[nslab + 3:]
        c = lax.axis_index("c")
        s = lax.axis_index("s")
        wid = s * NC + c
        base = wid * epw
        pltpu.sync_copy(zeros.at[pl.ds(s * npt, npt)],
                        shared.at[pl.ds(s * npt, npt)])
        plsc.subcore_barrier()

        for ef, si in zip(efs, slabs):
            off = si * (epw * NW) + wid * epw
            cps = [pltpu.async_copy(
                dsti.at[pl.ds(off + j * ch, ch)], idxd.at[j], isem)
                for j in range(nch)]
            for cp in cps:
                cp.wait()

            def body(j, carry):
                pltpu.sync_copy(ef.at[pl.ds(base + j * ch, ch)], buf)
                pltpu.sync_copy(buf, shared.at[idxd.at[j]], add=True)
                return carry

            lax.fori_loop(0, nch, body, 0)
        plsc.subcore_barrier()
        pltpu.sync_copy(shared.at[pl.ds(s * npt, npt)],
                        out.at[c].at[pl.ds(s * npt, npt)])

    return scatter


def kernel(x, edge_index, edge_sh, edge_attr, W_pre0, b_pre0, W_pre1,
           W_node0, b_node0, W_node1, Wg1, bg1, Wg2, bg2, Wf1, Wf2,
           Wl1, Wl2, Wo0, bo0, Wo1):
    n = x.shape[0]
    e = edge_index.shape[1]
    ch = 80
    n_pad = ((n + 8 * NS - 1) // (8 * NS)) * 8 * NS
    assert e % NW == 0 and (e // NW) % ch == 0

    f32 = jnp.float32
    perm = jnp.asarray(_PERM_NP)
    i3 = jnp.eye(3, dtype=f32)
    s32 = 1.0 / math.sqrt(32.0)

    m_pre1 = (W_pre1[:, None, None, :] * i3[None, :, :, None]
              * s32).reshape(96, 96)
    m_node1 = (i3[:, None, :, None] * W_node1[None, :, None, :]
               * s32).reshape(96, 96)
    wp = (Wl1[:32] + Wl1[32:64]) / math.sqrt(96.0)
    wc = Wl1[64:96] / math.sqrt(96.0)
    wl2 = Wl2 / math.sqrt(32.0)
    wf1 = Wf1 / math.sqrt(16.0)
    wf2 = Wf2 / math.sqrt(32.0)
    wa = jnp.zeros((48, 64), f32)
    wa = wa.at[:32, :32].set(wc).at[32:48, 32:64].set(wf1)
    col_scale = jnp.concatenate([jnp.full((32,), _C0, f32),
                                 jnp.full((32,), _C1, f32)])
    wb = jnp.zeros((64, 128), f32)
    wb = wb.at[:32, :64].set(wl2 * col_scale[None, :]).at[32:64, 64:128].set(wf2)
    m_out = (i3[:, None, None, :] * Wo1[None, :, :, None]
             * s32).reshape(96, 96)
    wo0 = Wo0 * s32
    bpre0 = b_pre0.reshape(1, 32)
    bnode0 = b_node0.reshape(1, 32)
    bg1r = bg1.reshape(1, 64)
    bg2r = bg2.reshape(1, 64)
    bo0r = bo0.reshape(1, 32)

    bn = 2000
    gn = n // bn
    be = 6400
    ge = e // be

    def full(shape):
        return pl.BlockSpec(shape, lambda i: (0,) * len(shape))

    t1, t2 = pl.pallas_call(
        _node_body,
        grid=(gn,),
        in_specs=[
            pl.BlockSpec((bn, 128), lambda i: (i, 0)),
            full((96, 96)), full((32, 32)), full((1, 32)), full((96, 96)),
            full((32, 32)), full((64, 64)), full((1, 64)), full((64, 64)),
            full((1, 64)), full((32, 32)), full((1, 32)), full((96, 96)),
        ],
        out_specs=[pl.BlockSpec((bn, 128), lambda i: (i, 0)),
                   pl.BlockSpec((bn, 256), lambda i: (i, 0))],
        out_shape=[jax.ShapeDtypeStruct((n, 128), f32),
                   jax.ShapeDtypeStruct((n, 256), f32)],
        compiler_params=pltpu.CompilerParams(
            dimension_semantics=("arbitrary",)),
    )(x, perm, W_pre0 * s32, bpre0, m_pre1, wp, Wg1, bg1r, Wg2, bg2r,
      W_node0 * s32, bnode0, m_node1)

    nslab = 5
    es = e // nslab
    epw = es // NW
    nch = epw // ch
    assert es % NW == 0 and epw % ch == 0
    ges = es // be
    dstf = edge_index[0]
    srcf = edge_index[1]
    ef_slabs = []
    for si in range(nslab):
        g1, g2 = _make_gather(n, es, ch, si)(t1, t2, dstf, srcf)
        base_blk = si * ges
        efs = pl.pallas_call(
            _edge_body,
            grid=(ges,),
            in_specs=[
                pl.BlockSpec((be, 128), lambda i: (i, 0)),
                pl.BlockSpec((be, 256), lambda i: (i, 0)),
                pl.BlockSpec((be, 16), lambda i, b=base_blk: (b + i, 0)),
                pl.BlockSpec((be, 1), lambda i, b=base_blk: (b + i, 0)),
                full((48, 64)), full((64, 128)),
            ],
            out_specs=pl.BlockSpec((be, 128), lambda i: (i, 0)),
            out_shape=jax.ShapeDtypeStruct((es, 128), f32),
            compiler_params=pltpu.CompilerParams(
                dimension_semantics=("arbitrary",)),
        )(g1, g2, edge_attr, edge_sh, wa, wb)
        ef_slabs.append(efs)

    zeros = jnp.zeros((n_pad, 128), f32)
    pacc_a = _make_scatter(n_pad, es, ch, (0, 1, 2, 3))(
        ef_slabs[0], ef_slabs[1], ef_slabs[2], ef_slabs[3], dstf, zeros)
    pacc_b = _make_scatter(n_pad, es, ch, (4,))(
        ef_slabs[4], dstf, zeros)

    out = pl.pallas_call(
        _final_body,
        grid=(gn,),
        in_specs=[
            pl.BlockSpec((NC, bn, 128), lambda i: (0, i, 0)),
            pl.BlockSpec((NC, bn, 128), lambda i: (0, i, 0)),
            pl.BlockSpec((bn, 256), lambda i: (i, 0)),
            full((32, 32)), full((1, 32)), full((96, 96)),
        ],
        out_specs=pl.BlockSpec((bn, 128), lambda i: (i, 0)),
        out_shape=jax.ShapeDtypeStruct((n, 128), f32),
        compiler_params=pltpu.CompilerParams(
            dimension_semantics=("arbitrary",)),
    )(pacc_a, pacc_b, t2, wo0, bo0r, m_out)

    return out


# speedup vs baseline: 1.4017x; 1.4017x over previous
import functools
import math

import numpy as np
import jax
import jax.numpy as jnp
from jax import lax
from jax.experimental import pallas as pl
from jax.experimental.pallas import tpu as pltpu
from jax.experimental.pallas import tpu_sc as plsc

NC = 2
NS = 16
NW = NC * NS

_PERM_NP = np.zeros((96, 96), np.float32)
for _u in range(32):
    for _i in range(3):
        _PERM_NP[3 * _u + _i, 32 * _i + _u] = 1.0

_LOG2 = math.log(2.0)
_C_IP = 1.0 / (3.0 * math.sqrt(3.0))
_C0 = math.sqrt(0.5)
_C1 = math.sqrt(1.5)


def _ssp(v):
    return jnp.maximum(v, 0.0) + jnp.log1p(jnp.exp(-jnp.abs(v))) - _LOG2


def _dot(a, b):
    return jax.lax.dot_general(a, b, (((1,), (0,)), ((), ())),
                               preferred_element_type=jnp.float32,
                               precision=jax.lax.Precision.HIGHEST)


def _node_body(x_ref, perm_ref, wpre0_ref, bpre0_ref, mpre1_ref, wp_ref,
               wg1_ref, bg1_ref, wg2_ref, bg2_ref, wnode0_ref, bnode0_ref,
               mnode1_ref, t1_ref, t2_ref):
    x = x_ref[...]
    x0 = x[:, :32]
    x1u = x[:, 32:]
    x1im = _dot(x1u, perm_ref[...])
    pre0 = _dot(x0, wpre0_ref[...]) + bpre0_ref[...]
    pre1im = _dot(x1u, mpre1_ref[...])
    p = _dot(pre0, wp_ref[...])
    n1 = jnp.sqrt(x1im[:, :32] ** 2 + x1im[:, 32:64] ** 2
                  + x1im[:, 64:96] ** 2 + 1e-12)
    f0 = jnp.concatenate([x0, n1], axis=1)
    h = _dot(f0, wg1_ref[...]) + bg1_ref[...]
    g = _dot(h * jax.nn.sigmoid(h), wg2_ref[...]) + bg2_ref[...]
    g1 = g[:, 32:64]
    g1t = jnp.concatenate([g1, g1, g1], axis=1)
    xg1im = x1im * g1t
    xl0 = _dot(g[:, :32], wnode0_ref[...]) + bnode0_ref[...]
    xl1im = _dot(xg1im, mnode1_ref[...])
    t1_ref[...] = jnp.concatenate([p, pre1im], axis=1)
    t2_ref[...] = jnp.concatenate(
        [pre1im, xl0, xl1im, jnp.zeros_like(xl0)], axis=1)


def _dot0(a, b):
    return jax.lax.dot_general(a, b, (((0,), (0,)), ((), ())),
                               preferred_element_type=jnp.float32,
                               precision=jax.lax.Precision.HIGHEST)


def _edge_body(g1_ref, g2_ref, attrt_ref, sht_ref, wa_ref, wb_ref, ef_ref):
    g1 = g1_ref[...]
    g2 = g2_ref[...]
    pd = g1[:, :32]
    pd1 = g1[:, 32:128]
    ps1 = g2[:, :96]
    x0s = g2[:, 96:128]
    x1s = g2[:, 128:224]
    prod = pd1 * ps1
    ip1 = (prod[:, :32] + prod[:, 32:64] + prod[:, 64:96]) * _C_IP
    u = jnp.concatenate([ip1, attrt_ref[...].T], axis=1)
    huv = _dot(u, wa_ref[...])
    hs = _ssp(huv[:, :32] + pd)
    hf = _ssp(huv[:, 32:64])
    z = jnp.concatenate([hs, hf], axis=1)
    wz = _dot(z, wb_ref[...])
    w = wz[:, :64] * wz[:, 64:128] * sht_ref[...].T
    w1 = w[:, 32:64]
    ef_ref[:, :32] = x0s * w[:, :32]
    ef_ref[:, 32:128] = x1s * jnp.concatenate([w1, w1, w1], axis=1)


def _final_body(pa_ref, pb_ref, t2_ref, wo0_ref, bo0_ref, mout_ref, o_ref):
    pa = pa_ref[...]
    pb = pb_ref[...]
    t2 = t2_ref[...]
    p0 = pa[0] + pa[1]
    p1 = pb[0] + pb[1]
    acc0 = p0[:, :32] + p1[:, :32] + t2[:, 96:128]
    acc1 = p0[:, 32:] + p1[:, 32:] + t2[:, 128:224]
    o0 = _dot(acc0, wo0_ref[...]) + bo0_ref[...]
    o1 = _dot(acc1, mout_ref[...])
    o_ref[...] = jnp.concatenate([o0, o1], axis=1)


def _make_gather(n, es, ch, si):
    nch = (es // NW) // ch
    epw = es // NW
    mesh = plsc.VectorSubcoreMesh(core_axis_name="c", subcore_axis_name="s")

    @functools.partial(
        pl.kernel,
        out_type=[jax.ShapeDtypeStruct((es, 128), jnp.float32),
                  jax.ShapeDtypeStruct((es, 256), jnp.float32)],
        mesh=mesh,
        scratch_types=[
            pltpu.VMEM((nch, ch), jnp.int32),
            pltpu.VMEM((nch, ch), jnp.int32),
            pltpu.VMEM((ch, 128), jnp.float32),
            pltpu.VMEM((ch, 256), jnp.float32),
            pltpu.SemaphoreType.DMA,
            pltpu.SemaphoreType.DMA,
        ],
    )
    def gather(t1, t2, dsti, srci, g1, g2, idxd, idxs, b1, b2, sem1, sem2):
        c = lax.axis_index("c")
        s = lax.axis_index("s")
        wid = s * NC + c
        base = wid * epw
        pltpu.sync_copy(dsti.at[si * NW + wid], idxd)
        pltpu.sync_copy(srci.at[si * NW + wid], idxs)

        def body(j, carry):
            cp1 = pltpu.async_copy(t1.at[idxd.at[j]], b1, sem1)
            cp2 = pltpu.async_copy(t2.at[idxs.at[j]], b2, sem2)
            cp1.wait()
            cp2.wait()
            pltpu.sync_copy(b1, g1.at[pl.ds(base + j * ch, ch)])
            pltpu.sync_copy(b2, g2.at[pl.ds(base + j * ch, ch)])
            return carry

        lax.fori_loop(0, nch, body, 0)

    return gather


def _make_scatter(n_pad, es, ch, slabs):
    nch = (es // NW) // ch
    epw = es // NW
    npt = n_pad // NS
    nslab = len(slabs)
    mesh = plsc.VectorSubcoreMesh(core_axis_name="c", subcore_axis_name="s")

    @functools.partial(
        pl.kernel,
        out_type=jax.ShapeDtypeStruct((NC, n_pad, 128), jnp.float32),
        mesh=mesh,
        scratch_types=[
            pltpu.VMEM((nch, ch), jnp.int32),
            pltpu.VMEM((ch, 128), jnp.float32),
            pltpu.VMEM_SHARED((n_pad, 128), jnp.float32),
        ],
    )
    def scatter(*refs):
        efs = refs[:nslab]
        dsti = refs[nslab]
        zeros = refs[nslab + 1]
        out = refs[nslab + 2]
        idxd, buf, shared = refs[nslab + 3:]
        c = lax.axis_index("c")
        s = lax.axis_index("s")
        wid = s * NC + c
        base = wid * epw
        pltpu.sync_copy(zeros.at[pl.ds(s * npt, npt)],
                        shared.at[pl.ds(s * npt, npt)])
        plsc.subcore_barrier()

        for ef, si in zip(efs, slabs):
            pltpu.sync_copy(dsti.at[si * NW + wid], idxd)

            def body(j, carry):
                pltpu.sync_copy(ef.at[pl.ds(base + j * ch, ch)], buf)
                pltpu.sync_copy(buf, shared.at[idxd.at[j]], add=True)
                return carry

            lax.fori_loop(0, nch, body, 0)
        plsc.subcore_barrier()
        pltpu.sync_copy(shared.at[pl.ds(s * npt, npt)],
                        out.at[c].at[pl.ds(s * npt, npt)])

    return scatter


def kernel(x, edge_index, edge_sh, edge_attr, W_pre0, b_pre0, W_pre1,
           W_node0, b_node0, W_node1, Wg1, bg1, Wg2, bg2, Wf1, Wf2,
           Wl1, Wl2, Wo0, bo0, Wo1):
    n = x.shape[0]
    e = edge_index.shape[1]
    ch = 80
    n_pad = ((n + 8 * NS - 1) // (8 * NS)) * 8 * NS
    assert e % NW == 0 and (e // NW) % ch == 0

    f32 = jnp.float32
    perm = jnp.asarray(_PERM_NP)
    i3 = jnp.eye(3, dtype=f32)
    s32 = 1.0 / math.sqrt(32.0)

    m_pre1 = (W_pre1[:, None, None, :] * i3[None, :, :, None]
              * s32).reshape(96, 96)
    m_node1 = (i3[:, None, :, None] * W_node1[None, :, None, :]
               * s32).reshape(96, 96)
    wp = (Wl1[:32] + Wl1[32:64]) / math.sqrt(96.0)
    wc = Wl1[64:96] / math.sqrt(96.0)
    wl2 = Wl2 / math.sqrt(32.0)
    wf1 = Wf1 / math.sqrt(16.0)
    wf2 = Wf2 / math.sqrt(32.0)
    wa = jnp.zeros((48, 64), f32)
    wa = wa.at[:32, :32].set(wc).at[32:48, 32:64].set(wf1)
    col_scale = jnp.concatenate([jnp.full((32,), _C0, f32),
                                 jnp.full((32,), _C1, f32)])
    wb = jnp.zeros((64, 128), f32)
    wb = wb.at[:32, :64].set(wl2 * col_scale[None, :]).at[32:64, 64:128].set(wf2)
    m_out = (i3[:, None, None, :] * Wo1[None, :, :, None]
             * s32).reshape(96, 96)
    wo0 = Wo0 * s32
    bpre0 = b_pre0.reshape(1, 32)
    bnode0 = b_node0.reshape(1, 32)
    bg1r = bg1.reshape(1, 64)
    bg2r = bg2.reshape(1, 64)
    bo0r = bo0.reshape(1, 32)

    bn = 2000
    gn = n // bn
    be = 6400
    ge = e // be

    def full(shape):
        return pl.BlockSpec(shape, lambda i: (0,) * len(shape))

    t1, t2 = pl.pallas_call(
        _node_body,
        grid=(gn,),
        in_specs=[
            pl.BlockSpec((bn, 128), lambda i: (i, 0)),
            full((96, 96)), full((32, 32)), full((1, 32)), full((96, 96)),
            full((32, 32)), full((64, 64)), full((1, 64)), full((64, 64)),
            full((1, 64)), full((32, 32)), full((1, 32)), full((96, 96)),
        ],
        out_specs=[pl.BlockSpec((bn, 128), lambda i: (i, 0)),
                   pl.BlockSpec((bn, 256), lambda i: (i, 0))],
        out_shape=[jax.ShapeDtypeStruct((n, 128), f32),
                   jax.ShapeDtypeStruct((n, 256), f32)],
        compiler_params=pltpu.CompilerParams(
            dimension_semantics=("arbitrary",)),
    )(x, perm, W_pre0 * s32, bpre0, m_pre1, wp, Wg1, bg1r, Wg2, bg2r,
      W_node0 * s32, bnode0, m_node1)

    nslab = 5
    es = e // nslab
    epw = es // NW
    nch = epw // ch
    assert es % NW == 0 and epw % ch == 0
    ges = es // be
    dst3 = edge_index[0].reshape(nslab * NW, nch, ch)
    src3 = edge_index[1].reshape(nslab * NW, nch, ch)
    ef_slabs = []
    for si in range(nslab):
        g1, g2 = _make_gather(n, es, ch, si)(t1, t2, dst3, src3)
        base_blk = si * ges
        efs = pl.pallas_call(
            _edge_body,
            grid=(ges,),
            in_specs=[
                pl.BlockSpec((be, 128), lambda i: (i, 0)),
                pl.BlockSpec((be, 256), lambda i: (i, 0)),
                pl.BlockSpec((16, be), lambda i, b=base_blk: (0, b + i)),
                pl.BlockSpec((1, be), lambda i, b=base_blk: (0, b + i)),
                full((48, 64)), full((64, 128)),
            ],
            out_specs=pl.BlockSpec((be, 128), lambda i: (i, 0)),
            out_shape=jax.ShapeDtypeStruct((es, 128), f32),
            compiler_params=pltpu.CompilerParams(
                dimension_semantics=("arbitrary",)),
        )(g1, g2, edge_attr.T, edge_sh.reshape(1, e), wa, wb)
        ef_slabs.append(efs)

    zeros = jnp.zeros((n_pad, 128), f32)
    pacc_a = _make_scatter(n_pad, es, ch, (0, 1, 2))(
        ef_slabs[0], ef_slabs[1], ef_slabs[2], dst3, zeros)
    pacc_b = _make_scatter(n_pad, es, ch, (3, 4))(
        ef_slabs[3], ef_slabs[4], dst3, zeros)

    out = pl.pallas_call(
        _final_body,
        grid=(gn,),
        in_specs=[
            pl.BlockSpec((NC, bn, 128), lambda i: (0, i, 0)),
            pl.BlockSpec((NC, bn, 128), lambda i: (0, i, 0)),
            pl.BlockSpec((bn, 256), lambda i: (i, 0)),
            full((32, 32)), full((1, 32)), full((96, 96)),
        ],
        out_specs=pl.BlockSpec((bn, 128), lambda i: (i, 0)),
        out_shape=jax.ShapeDtypeStruct((n, 128), f32),
        compiler_params=pltpu.CompilerParams(
            dimension_semantics=("arbitrary",)),
    )(pacc_a, pacc_b, t2, wo0, bo0r, m_out)

    return out


# speedup vs baseline: 1.7006x; 1.2133x over previous
import functools
import math

import numpy as np
import jax
import jax.numpy as jnp
from jax import lax
from jax.experimental import pallas as pl
from jax.experimental.pallas import tpu as pltpu
from jax.experimental.pallas import tpu_sc as plsc

NC = 2
NS = 16
NW = NC * NS

_PERM_NP = np.zeros((96, 96), np.float32)
for _u in range(32):
    for _i in range(3):
        _PERM_NP[3 * _u + _i, 32 * _i + _u] = 1.0

_LOG2 = math.log(2.0)
_C_IP = 1.0 / (3.0 * math.sqrt(3.0))
_C0 = math.sqrt(0.5)
_C1 = math.sqrt(1.5)


def _ssp(v):
    return jnp.maximum(v, 0.0) + jnp.log1p(jnp.exp(-jnp.abs(v))) - _LOG2


def _dot(a, b):
    return jax.lax.dot_general(a, b, (((1,), (0,)), ((), ())),
                               preferred_element_type=jnp.float32,
                               precision=jax.lax.Precision.HIGHEST)


def _node_body(x_ref, perm_ref, wpre0_ref, bpre0_ref, mpre1_ref, wp_ref,
               wg1_ref, bg1_ref, wg2_ref, bg2_ref, wnode0_ref, bnode0_ref,
               mnode1_ref, t1_ref, t3_ref):
    x = x_ref[...]
    x0 = x[:, :32]
    x1u = x[:, 32:]
    x1im = _dot(x1u, perm_ref[...])
    pre0 = _dot(x0, wpre0_ref[...]) + bpre0_ref[...]
    pre1im = _dot(x1u, mpre1_ref[...])
    p = _dot(pre0, wp_ref[...])
    n1 = jnp.sqrt(x1im[:, :32] ** 2 + x1im[:, 32:64] ** 2
                  + x1im[:, 64:96] ** 2 + 1e-12)
    f0 = jnp.concatenate([x0, n1], axis=1)
    h = _dot(f0, wg1_ref[...]) + bg1_ref[...]
    g = _dot(h * jax.nn.sigmoid(h), wg2_ref[...]) + bg2_ref[...]
    g1 = g[:, 32:64]
    g1t = jnp.concatenate([g1, g1, g1], axis=1)
    xg1im = x1im * g1t
    xl0 = _dot(g[:, :32], wnode0_ref[...]) + bnode0_ref[...]
    xl1im = _dot(xg1im, mnode1_ref[...])
    t1_ref[...] = jnp.concatenate([p, pre1im], axis=1)
    t3_ref[...] = jnp.concatenate([xl0, xl1im], axis=1)


def _edge_body(g1d_ref, g1s_ref, attrt_ref, sht_ref, wa_ref, wb_ref, w_ref):
    g1d = g1d_ref[...]
    g1s = g1s_ref[...]
    ip1 = (g1d[:, 32:64] * g1s[:, 32:64]
           + g1d[:, 64:96] * g1s[:, 64:96]
           + g1d[:, 96:128] * g1s[:, 96:128]) * _C_IP
    u = jnp.concatenate([ip1, attrt_ref[...].T], axis=1)
    huv = _dot(u, wa_ref[...])
    hs = _ssp(huv[:, :32] + g1d[:, :32])
    hf = _ssp(huv[:, 32:64])
    z = jnp.concatenate([hs, hf], axis=1)
    wz = _dot(z, wb_ref[...])
    w_ref[...] = wz[:, :64] * wz[:, 64:128] * sht_ref[...].T


def _final_body(pa_ref, pb_ref, t3_ref, wo0_ref, bo0_ref, mout_ref, o_ref):
    pa = pa_ref[...]
    pb = pb_ref[...]
    t3 = t3_ref[...]
    p0 = pa[0] + pa[1]
    p1 = pb[0] + pb[1]
    acc0 = p0[:, :32] + p1[:, :32] + t3[:, :32]
    acc1 = p0[:, 32:] + p1[:, 32:] + t3[:, 32:128]
    o0 = _dot(acc0, wo0_ref[...]) + bo0_ref[...]
    o1 = _dot(acc1, mout_ref[...])
    o_ref[...] = jnp.concatenate([o0, o1], axis=1)


def _make_gather(n, es, ch, si):
    nch = (es // NW) // ch
    epw = es // NW
    mesh = plsc.VectorSubcoreMesh(core_axis_name="c", subcore_axis_name="s")

    @functools.partial(
        pl.kernel,
        out_type=[jax.ShapeDtypeStruct((es, 128), jnp.float32),
                  jax.ShapeDtypeStruct((es, 128), jnp.float32)],
        mesh=mesh,
        scratch_types=[
            pltpu.VMEM((nch, ch), jnp.int32),
            pltpu.VMEM((nch, ch), jnp.int32),
            pltpu.VMEM((ch, 128), jnp.float32),
            pltpu.VMEM((ch, 128), jnp.float32),
            pltpu.SemaphoreType.DMA,
            pltpu.SemaphoreType.DMA,
        ],
    )
    def gather(t1, dsti, srci, g1d, g1s, idxd, idxs, b1, b2, sem1, sem2):
        c = lax.axis_index("c")
        s = lax.axis_index("s")
        wid = s * NC + c
        base = wid * epw
        pltpu.sync_copy(dsti.at[si * NW + wid], idxd)
        pltpu.sync_copy(srci.at[si * NW + wid], idxs)

        def body(j, carry):
            cp1 = pltpu.async_copy(t1.at[idxd.at[j]], b1, sem1)
            cp2 = pltpu.async_copy(t1.at[idxs.at[j]], b2, sem2)
            cp1.wait()
            cp2.wait()
            pltpu.sync_copy(b1, g1d.at[pl.ds(base + j * ch, ch)])
            pltpu.sync_copy(b2, g1s.at[pl.ds(base + j * ch, ch)])
            return carry

        lax.fori_loop(0, nch, body, 0)

    return gather


def _make_scatter(n_pad, es, ch, slabs):
    nch = (es // NW) // ch
    epw = es // NW
    npt = n_pad // NS
    nslab = len(slabs)
    mesh = plsc.VectorSubcoreMesh(core_axis_name="c", subcore_axis_name="s")

    @functools.partial(
        pl.kernel,
        out_type=jax.ShapeDtypeStruct((NC, n_pad, 128), jnp.float32),
        mesh=mesh,
        scratch_types=[
            pltpu.VMEM((nch, ch), jnp.int32),
            pltpu.VMEM((nch, ch), jnp.int32),
            pltpu.VMEM((ch, 64), jnp.float32),
            pltpu.VMEM((ch, 128), jnp.float32),
            pltpu.VMEM((ch, 128), jnp.float32),
            pltpu.VMEM_SHARED((n_pad, 128), jnp.float32),
            pltpu.SemaphoreType.DMA,
        ],
    )
    def scatter(*refs):
        ws = refs[:nslab]
        t3 = refs[nslab]
        dsti = refs[nslab + 1]
        srci = refs[nslab + 2]
        zeros = refs[nslab + 3]
        out = refs[nslab + 4]
        idxd, idxs, bufw, bufx, bufe, shared, sem = refs[nslab + 5:]
        c = lax.axis_index("c")
        s = lax.axis_index("s")
        wid = s * NC + c
        base = wid * epw
        pltpu.sync_copy(zeros.at[pl.ds(s * npt, npt)],
                        shared.at[pl.ds(s * npt, npt)])
        plsc.subcore_barrier()

        for wslab, si in zip(ws, slabs):
            pltpu.sync_copy(dsti.at[si * NW + wid], idxd)
            pltpu.sync_copy(srci.at[si * NW + wid], idxs)

            def body(j, carry):
                cpx = pltpu.async_copy(t3.at[idxs.at[j]], bufx, sem)
                pltpu.sync_copy(wslab.at[pl.ds(base + j * ch, ch)], bufw)
                cpx.wait()

                def edge(p, carry2):
                    wv = [bufw[p, pl.ds(16 * k, 16)] for k in range(4)]
                    sel = [0, 1, 2, 3, 2, 3, 2, 3]
                    for k in range(8):
                        bufe[p, pl.ds(16 * k, 16)] = (
                            bufx[p, pl.ds(16 * k, 16)] * wv[sel[k]])
                    return carry2

                lax.fori_loop(0, ch, edge, 0)
                pltpu.sync_copy(bufe, shared.at[idxd.at[j]], add=True)
                return carry

            lax.fori_loop(0, nch, body, 0)
        plsc.subcore_barrier()
        pltpu.sync_copy(shared.at[pl.ds(s * npt, npt)],
                        out.at[c].at[pl.ds(s * npt, npt)])

    return scatter


def kernel(x, edge_index, edge_sh, edge_attr, W_pre0, b_pre0, W_pre1,
           W_node0, b_node0, W_node1, Wg1, bg1, Wg2, bg2, Wf1, Wf2,
           Wl1, Wl2, Wo0, bo0, Wo1):
    n = x.shape[0]
    e = edge_index.shape[1]
    ch = 80
    n_pad = ((n + 8 * NS - 1) // (8 * NS)) * 8 * NS
    assert e % NW == 0 and (e // NW) % ch == 0

    f32 = jnp.float32
    perm = jnp.asarray(_PERM_NP)
    i3 = jnp.eye(3, dtype=f32)
    s32 = 1.0 / math.sqrt(32.0)

    m_pre1 = (W_pre1[:, None, None, :] * i3[None, :, :, None]
              * s32).reshape(96, 96)
    m_node1 = (i3[:, None, :, None] * W_node1[None, :, None, :]
               * s32).reshape(96, 96)
    wp = (Wl1[:32] + Wl1[32:64]) / math.sqrt(96.0)
    wc = Wl1[64:96] / math.sqrt(96.0)
    wl2 = Wl2 / math.sqrt(32.0)
    wf1 = Wf1 / math.sqrt(16.0)
    wf2 = Wf2 / math.sqrt(32.0)
    m_out = (i3[:, None, None, :] * Wo1[None, :, :, None]
             * s32).reshape(96, 96)
    wo0 = Wo0 * s32
    bpre0 = b_pre0.reshape(1, 32)
    bnode0 = b_node0.reshape(1, 32)
    bg1r = bg1.reshape(1, 64)
    bg2r = bg2.reshape(1, 64)
    bo0r = bo0.reshape(1, 32)

    wa = jnp.zeros((48, 64), f32)
    wa = wa.at[:32, :32].set(wc).at[32:48, 32:64].set(wf1)
    col_scale = jnp.concatenate([jnp.full((32,), _C0, f32),
                                 jnp.full((32,), _C1, f32)])
    wb = jnp.zeros((64, 128), f32)
    wb = wb.at[:32, :64].set(wl2 * col_scale[None, :]).at[32:64, 64:128].set(wf2)

    bn = 2000
    gn = n // bn
    be = 6400

    def full(shape):
        return pl.BlockSpec(shape, lambda i: (0,) * len(shape))

    t1, t3 = pl.pallas_call(
        _node_body,
        grid=(gn,),
        in_specs=[
            pl.BlockSpec((bn, 128), lambda i: (i, 0)),
            full((96, 96)), full((32, 32)), full((1, 32)), full((96, 96)),
            full((32, 32)), full((64, 64)), full((1, 64)), full((64, 64)),
            full((1, 64)), full((32, 32)), full((1, 32)), full((96, 96)),
        ],
        out_specs=[pl.BlockSpec((bn, 128), lambda i: (i, 0)),
                   pl.BlockSpec((bn, 128), lambda i: (i, 0))],
        out_shape=[jax.ShapeDtypeStruct((n, 128), f32),
                   jax.ShapeDtypeStruct((n, 128), f32)],
        compiler_params=pltpu.CompilerParams(
            dimension_semantics=("arbitrary",)),
    )(x, perm, W_pre0 * s32, bpre0, m_pre1, wp, Wg1, bg1r, Wg2, bg2r,
      W_node0 * s32, bnode0, m_node1)

    nslab = 5
    es = e // nslab
    epw = es // NW
    nch = epw // ch
    assert es % NW == 0 and epw % ch == 0 and es % be == 0
    ges = es // be
    dst3 = edge_index[0].reshape(nslab * NW, nch, ch)
    src3 = edge_index[1].reshape(nslab * NW, nch, ch)
    attr_t = edge_attr.T
    sh_t = edge_sh.reshape(1, e)
    w_slabs = []
    for si in range(nslab):
        g1d, g1s = _make_gather(n, es, ch, si)(t1, dst3, src3)
        base_blk = si * ges
        ws = pl.pallas_call(
            _edge_body,
            grid=(ges,),
            in_specs=[
                pl.BlockSpec((be, 128), lambda i: (i, 0)),
                pl.BlockSpec((be, 128), lambda i: (i, 0)),
                pl.BlockSpec((16, be), lambda i, b=base_blk: (0, b + i)),
                pl.BlockSpec((1, be), lambda i, b=base_blk: (0, b + i)),
                full((48, 64)), full((64, 128)),
            ],
            out_specs=pl.BlockSpec((be, 64), lambda i: (i, 0)),
            out_shape=jax.ShapeDtypeStruct((es, 64), f32),
            compiler_params=pltpu.CompilerParams(
                dimension_semantics=("arbitrary",)),
        )(g1d, g1s, attr_t, sh_t, wa, wb)
        w_slabs.append(ws)

    zeros = jnp.zeros((n_pad, 128), f32)
    pacc_a = _make_scatter(n_pad, es, ch, (0, 1, 2))(
        w_slabs[0], w_slabs[1], w_slabs[2], t3, dst3, src3, zeros)
    pacc_b = _make_scatter(n_pad, es, ch, (3, 4))(
        w_slabs[3], w_slabs[4], t3, dst3, src3, zeros)

    out = pl.pallas_call(
        _final_body,
        grid=(gn,),
        in_specs=[
            pl.BlockSpec((NC, bn, 128), lambda i: (0, i, 0)),
            pl.BlockSpec((NC, bn, 128), lambda i: (0, i, 0)),
            pl.BlockSpec((bn, 128), lambda i: (i, 0)),
            full((32, 32)), full((1, 32)), full((96, 96)),
        ],
        out_specs=pl.BlockSpec((bn, 128), lambda i: (i, 0)),
        out_shape=jax.ShapeDtypeStruct((n, 128), f32),
        compiler_params=pltpu.CompilerParams(
            dimension_semantics=("arbitrary",)),
    )(pacc_a, pacc_b, t3, wo0, bo0r, m_out)

    return out


# speedup vs baseline: 1.7866x; 1.0505x over previous
import functools
import math

import numpy as np
import jax
import jax.numpy as jnp
from jax import lax
from jax.experimental import pallas as pl
from jax.experimental.pallas import tpu as pltpu
from jax.experimental.pallas import tpu_sc as plsc

NC = 2
NS = 16
NW = NC * NS

_PERM_NP = np.zeros((96, 96), np.float32)
for _u in range(32):
    for _i in range(3):
        _PERM_NP[3 * _u + _i, 32 * _i + _u] = 1.0

_LOG2 = math.log(2.0)
_C_IP = 1.0 / (3.0 * math.sqrt(3.0))
_C0 = math.sqrt(0.5)
_C1 = math.sqrt(1.5)


def _ssp(v):
    return jnp.maximum(v, 0.0) + jnp.log1p(jnp.exp(-jnp.abs(v))) - _LOG2


def _dot(a, b):
    return jax.lax.dot_general(a, b, (((1,), (0,)), ((), ())),
                               preferred_element_type=jnp.float32,
                               precision=jax.lax.Precision.HIGHEST)


def _node_body(x_ref, perm_ref, wpre0_ref, bpre0_ref, mpre1_ref, wp_ref,
               wg1_ref, bg1_ref, wg2_ref, bg2_ref, wnode0_ref, bnode0_ref,
               mnode1_ref, t1_ref, t3_ref):
    x = x_ref[...]
    x0 = x[:, :32]
    x1u = x[:, 32:]
    x1im = _dot(x1u, perm_ref[...])
    pre0 = _dot(x0, wpre0_ref[...]) + bpre0_ref[...]
    pre1im = _dot(x1u, mpre1_ref[...])
    p = _dot(pre0, wp_ref[...])
    n1 = jnp.sqrt(x1im[:, :32] ** 2 + x1im[:, 32:64] ** 2
                  + x1im[:, 64:96] ** 2 + 1e-12)
    f0 = jnp.concatenate([x0, n1], axis=1)
    h = _dot(f0, wg1_ref[...]) + bg1_ref[...]
    g = _dot(h * jax.nn.sigmoid(h), wg2_ref[...]) + bg2_ref[...]
    g1 = g[:, 32:64]
    g1t = jnp.concatenate([g1, g1, g1], axis=1)
    xg1im = x1im * g1t
    xl0 = _dot(g[:, :32], wnode0_ref[...]) + bnode0_ref[...]
    xl1im = _dot(xg1im, mnode1_ref[...])
    t1_ref[...] = jnp.concatenate([p, pre1im], axis=1)
    t3_ref[...] = jnp.concatenate([xl0, xl1im], axis=1)


def _edge_body(g1d_ref, g1s_ref, attrt_ref, sht_ref, wa_ref, wb_ref, w_ref):
    g1d = g1d_ref[...]
    g1s = g1s_ref[...]
    ip1 = (g1d[:, 32:64] * g1s[:, 32:64]
           + g1d[:, 64:96] * g1s[:, 64:96]
           + g1d[:, 96:128] * g1s[:, 96:128]) * _C_IP
    u = jnp.concatenate([ip1, attrt_ref[...].T], axis=1)
    huv = _dot(u, wa_ref[...])
    hs = _ssp(huv[:, :32] + g1d[:, :32])
    hf = _ssp(huv[:, 32:64])
    z = jnp.concatenate([hs, hf], axis=1)
    wz = _dot(z, wb_ref[...])
    w_ref[...] = wz[:, :64] * wz[:, 64:128] * sht_ref[...].T


def _final_body(pa_ref, pb_ref, pc_ref, t3_ref, wo0_ref, bo0_ref,
                mout_ref, o_ref):
    pa = pa_ref[...]
    pb = pb_ref[...]
    pc = pc_ref[...]
    t3 = t3_ref[...]
    p0 = pa[0] + pa[1] + pc[0]
    p1 = pb[0] + pb[1] + pc[1]
    acc0 = p0[:, :32] + p1[:, :32] + t3[:, :32]
    acc1 = p0[:, 32:] + p1[:, 32:] + t3[:, 32:128]
    o0 = _dot(acc0, wo0_ref[...]) + bo0_ref[...]
    o1 = _dot(acc1, mout_ref[...])
    o_ref[...] = jnp.concatenate([o0, o1], axis=1)


def _make_gather(n, es, ch, si):
    nch = (es // NW) // ch
    epw = es // NW
    mesh = plsc.VectorSubcoreMesh(core_axis_name="c", subcore_axis_name="s")

    @functools.partial(
        pl.kernel,
        out_type=[jax.ShapeDtypeStruct((es, 128), jnp.float32),
                  jax.ShapeDtypeStruct((es, 128), jnp.float32)],
        mesh=mesh,
        scratch_types=[
            pltpu.VMEM((nch, ch), jnp.int32),
            pltpu.VMEM((nch, ch), jnp.int32),
            pltpu.VMEM((ch, 128), jnp.float32),
            pltpu.VMEM((ch, 128), jnp.float32),
            pltpu.SemaphoreType.DMA,
            pltpu.SemaphoreType.DMA,
        ],
    )
    def gather(t1, dsti, srci, g1d, g1s, idxd, idxs, b1, b2, sem1, sem2):
        c = lax.axis_index("c")
        s = lax.axis_index("s")
        wid = s * NC + c
        base = wid * epw
        pltpu.sync_copy(dsti.at[si * NW + wid], idxd)
        pltpu.sync_copy(srci.at[si * NW + wid], idxs)

        def body(j, carry):
            cp1 = pltpu.async_copy(t1.at[idxd.at[j]], b1, sem1)
            cp2 = pltpu.async_copy(t1.at[idxs.at[j]], b2, sem2)
            cp1.wait()
            cp2.wait()
            pltpu.sync_copy(b1, g1d.at[pl.ds(base + j * ch, ch)])
            pltpu.sync_copy(b2, g1s.at[pl.ds(base + j * ch, ch)])
            return carry

        lax.fori_loop(0, nch, body, 0)

    return gather


def _make_scatter(n_pad, es, ch, slabs):
    nch = (es // NW) // ch
    epw = es // NW
    npt = n_pad // NS
    nslab = len(slabs)
    mesh = plsc.VectorSubcoreMesh(core_axis_name="c", subcore_axis_name="s")

    @functools.partial(
        pl.kernel,
        out_type=jax.ShapeDtypeStruct((NC, n_pad, 128), jnp.float32),
        mesh=mesh,
        scratch_types=[
            pltpu.VMEM((nch, ch), jnp.int32),
            pltpu.VMEM((nch, ch), jnp.int32),
            pltpu.VMEM((ch, 64), jnp.float32),
            pltpu.VMEM((ch, 128), jnp.float32),
            pltpu.VMEM((ch, 128), jnp.float32),
            pltpu.VMEM_SHARED((n_pad, 128), jnp.float32),
            pltpu.SemaphoreType.DMA,
        ],
    )
    def scatter(*refs):
        ws = refs[:nslab]
        t3 = refs[nslab]
        dsti = refs[nslab + 1]
        srci = refs[nslab + 2]
        zeros = refs[nslab + 3]
        out = refs[nslab + 4]
        idxd, idxs, bufw, bufx, bufe, shared, sem = refs[nslab + 5:]
        c = lax.axis_index("c")
        s = lax.axis_index("s")
        wid = s * NC + c
        base = wid * epw
        pltpu.sync_copy(zeros.at[pl.ds(s * npt, npt)],
                        shared.at[pl.ds(s * npt, npt)])
        plsc.subcore_barrier()

        for wslab, si in zip(ws, slabs):
            pltpu.sync_copy(dsti.at[si * NW + wid], idxd)
            pltpu.sync_copy(srci.at[si * NW + wid], idxs)

            def body(j, carry):
                cpx = pltpu.async_copy(t3.at[idxs.at[j]], bufx, sem)
                pltpu.sync_copy(wslab.at[pl.ds(base + j * ch, ch)], bufw)
                cpx.wait()

                def edge(p, carry2):
                    wv = [bufw[p, pl.ds(16 * k, 16)] for k in range(4)]
                    sel = [0, 1, 2, 3, 2, 3, 2, 3]
                    for k in range(8):
                        bufe[p, pl.ds(16 * k, 16)] = (
                            bufx[p, pl.ds(16 * k, 16)] * wv[sel[k]])
                    return carry2

                lax.fori_loop(0, ch, edge, 0)
                pltpu.sync_copy(bufe, shared.at[idxd.at[j]], add=True)
                return carry

            lax.fori_loop(0, nch, body, 0)
        plsc.subcore_barrier()
        pltpu.sync_copy(shared.at[pl.ds(s * npt, npt)],
                        out.at[c].at[pl.ds(s * npt, npt)])

    return scatter


def kernel(x, edge_index, edge_sh, edge_attr, W_pre0, b_pre0, W_pre1,
           W_node0, b_node0, W_node1, Wg1, bg1, Wg2, bg2, Wf1, Wf2,
           Wl1, Wl2, Wo0, bo0, Wo1):
    n = x.shape[0]
    e = edge_index.shape[1]
    ch = 80
    n_pad = ((n + 8 * NS - 1) // (8 * NS)) * 8 * NS
    assert e % NW == 0 and (e // NW) % ch == 0

    f32 = jnp.float32
    perm = jnp.asarray(_PERM_NP)
    i3 = jnp.eye(3, dtype=f32)
    s32 = 1.0 / math.sqrt(32.0)

    m_pre1 = (W_pre1[:, None, None, :] * i3[None, :, :, None]
              * s32).reshape(96, 96)
    m_node1 = (i3[:, None, :, None] * W_node1[None, :, None, :]
               * s32).reshape(96, 96)
    wp = (Wl1[:32] + Wl1[32:64]) / math.sqrt(96.0)
    wc = Wl1[64:96] / math.sqrt(96.0)
    wl2 = Wl2 / math.sqrt(32.0)
    wf1 = Wf1 / math.sqrt(16.0)
    wf2 = Wf2 / math.sqrt(32.0)
    m_out = (i3[:, None, None, :] * Wo1[None, :, :, None]
             * s32).reshape(96, 96)
    wo0 = Wo0 * s32
    bpre0 = b_pre0.reshape(1, 32)
    bnode0 = b_node0.reshape(1, 32)
    bg1r = bg1.reshape(1, 64)
    bg2r = bg2.reshape(1, 64)
    bo0r = bo0.reshape(1, 32)

    wa = jnp.zeros((48, 64), f32)
    wa = wa.at[:32, :32].set(wc).at[32:48, 32:64].set(wf1)
    col_scale = jnp.concatenate([jnp.full((32,), _C0, f32),
                                 jnp.full((32,), _C1, f32)])
    wb = jnp.zeros((64, 128), f32)
    wb = wb.at[:32, :64].set(wl2 * col_scale[None, :]).at[32:64, 64:128].set(wf2)

    bn = 2000
    gn = n // bn
    be = 6400

    def full(shape):
        return pl.BlockSpec(shape, lambda i: (0,) * len(shape))

    t1, t3 = pl.pallas_call(
        _node_body,
        grid=(gn,),
        in_specs=[
            pl.BlockSpec((bn, 128), lambda i: (i, 0)),
            full((96, 96)), full((32, 32)), full((1, 32)), full((96, 96)),
            full((32, 32)), full((64, 64)), full((1, 64)), full((64, 64)),
            full((1, 64)), full((32, 32)), full((1, 32)), full((96, 96)),
        ],
        out_specs=[pl.BlockSpec((bn, 128), lambda i: (i, 0)),
                   pl.BlockSpec((bn, 128), lambda i: (i, 0))],
        out_shape=[jax.ShapeDtypeStruct((n, 128), f32),
                   jax.ShapeDtypeStruct((n, 128), f32)],
        compiler_params=pltpu.CompilerParams(
            dimension_semantics=("arbitrary",)),
    )(x, perm, W_pre0 * s32, bpre0, m_pre1, wp, Wg1, bg1r, Wg2, bg2r,
      W_node0 * s32, bnode0, m_node1)

    nslab = 5
    es = e // nslab
    epw = es // NW
    nch = epw // ch
    assert es % NW == 0 and epw % ch == 0 and es % be == 0
    ges = es // be
    dst3 = edge_index[0].reshape(nslab * NW, nch, ch)
    src3 = edge_index[1].reshape(nslab * NW, nch, ch)
    attr_t = edge_attr.T
    sh_t = edge_sh.reshape(1, e)
    w_slabs = []
    for si in range(nslab):
        g1d, g1s = _make_gather(n, es, ch, si)(t1, dst3, src3)
        base_blk = si * ges
        ws = pl.pallas_call(
            _edge_body,
            grid=(ges,),
            in_specs=[
                pl.BlockSpec((be, 128), lambda i: (i, 0)),
                pl.BlockSpec((be, 128), lambda i: (i, 0)),
                pl.BlockSpec((16, be), lambda i, b=base_blk: (0, b + i)),
                pl.BlockSpec((1, be), lambda i, b=base_blk: (0, b + i)),
                full((48, 64)), full((64, 128)),
            ],
            out_specs=pl.BlockSpec((be, 64), lambda i: (i, 0)),
            out_shape=jax.ShapeDtypeStruct((es, 64), f32),
            compiler_params=pltpu.CompilerParams(
                dimension_semantics=("arbitrary",)),
        )(g1d, g1s, attr_t, sh_t, wa, wb)
        w_slabs.append(ws)

    zeros = jnp.zeros((n_pad, 128), f32)
    pacc_a = _make_scatter(n_pad, es, ch, (0, 1))(
        w_slabs[0], w_slabs[1], t3, dst3, src3, zeros)
    pacc_b = _make_scatter(n_pad, es, ch, (2, 3))(
        w_slabs[2], w_slabs[3], t3, dst3, src3, zeros)
    pacc_c = _make_scatter(n_pad, es, ch, (4,))(
        w_slabs[4], t3, dst3, src3, zeros)

    out = pl.pallas_call(
        _final_body,
        grid=(gn,),
        in_specs=[
            pl.BlockSpec((NC, bn, 128), lambda i: (0, i, 0)),
            pl.BlockSpec((NC, bn, 128), lambda i: (0, i, 0)),
            pl.BlockSpec((NC, bn, 128), lambda i: (0, i, 0)),
            pl.BlockSpec((bn, 128), lambda i: (i, 0)),
            full((32, 32)), full((1, 32)), full((96, 96)),
        ],
        out_specs=pl.BlockSpec((bn, 128), lambda i: (i, 0)),
        out_shape=jax.ShapeDtypeStruct((n, 128), f32),
        compiler_params=pltpu.CompilerParams(
            dimension_semantics=("arbitrary",)),
    )(pacc_a, pacc_b, pacc_c, t3, wo0, bo0r, m_out)

    return out


# speedup vs baseline: 1.8396x; 1.0297x over previous
import functools
import math

import numpy as np
import jax
import jax.numpy as jnp
from jax import lax
from jax.experimental import pallas as pl
from jax.experimental.pallas import tpu as pltpu
from jax.experimental.pallas import tpu_sc as plsc

NC = 2
NS = 16
NW = NC * NS

_PERM_NP = np.zeros((96, 96), np.float32)
for _u in range(32):
    for _i in range(3):
        _PERM_NP[3 * _u + _i, 32 * _i + _u] = 1.0

_LOG2 = math.log(2.0)
_C_IP = 1.0 / (3.0 * math.sqrt(3.0))
_C0 = math.sqrt(0.5)
_C1 = math.sqrt(1.5)


def _ssp(v):
    return jnp.maximum(v, 0.0) + jnp.log1p(jnp.exp(-jnp.abs(v))) - _LOG2


def _dot(a, b):
    return jax.lax.dot_general(a, b, (((1,), (0,)), ((), ())),
                               preferred_element_type=jnp.float32,
                               precision=jax.lax.Precision.HIGHEST)


def _node_a_body(x_ref, perm_ref, wpre0_ref, bpre0_ref, mpre1_ref, wp_ref,
                 t1_ref):
    x = x_ref[...]
    x0 = x[:, :32]
    x1u = x[:, 32:]
    pre0 = _dot(x0, wpre0_ref[...]) + bpre0_ref[...]
    pre1im = _dot(x1u, mpre1_ref[...])
    p = _dot(pre0, wp_ref[...])
    t1_ref[...] = jnp.concatenate([p, pre1im], axis=1)


def _node_b_body(x_ref, perm_ref, wg1_ref, bg1_ref, wg2_ref, bg2_ref,
                 wnode0_ref, bnode0_ref, mnode1_ref, t3_ref):
    x = x_ref[...]
    x0 = x[:, :32]
    x1u = x[:, 32:]
    x1im = _dot(x1u, perm_ref[...])
    n1 = jnp.sqrt(x1im[:, :32] ** 2 + x1im[:, 32:64] ** 2
                  + x1im[:, 64:96] ** 2 + 1e-12)
    f0 = jnp.concatenate([x0, n1], axis=1)
    h = _dot(f0, wg1_ref[...]) + bg1_ref[...]
    g = _dot(h * jax.nn.sigmoid(h), wg2_ref[...]) + bg2_ref[...]
    g1 = g[:, 32:64]
    g1t = jnp.concatenate([g1, g1, g1], axis=1)
    xg1im = x1im * g1t
    xl0 = _dot(g[:, :32], wnode0_ref[...]) + bnode0_ref[...]
    xl1im = _dot(xg1im, mnode1_ref[...])
    t3_ref[...] = jnp.concatenate([xl0, xl1im], axis=1)


def _edge_body(g1d_ref, g1s_ref, attrt_ref, sht_ref, wa_ref, wb_ref, w_ref):
    g1d = g1d_ref[...]
    g1s = g1s_ref[...]
    ip1 = (g1d[:, 32:64] * g1s[:, 32:64]
           + g1d[:, 64:96] * g1s[:, 64:96]
           + g1d[:, 96:128] * g1s[:, 96:128]) * _C_IP
    u = jnp.concatenate([ip1, attrt_ref[...].T], axis=1)
    huv = _dot(u, wa_ref[...])
    hs = _ssp(huv[:, :32] + g1d[:, :32])
    hf = _ssp(huv[:, 32:64])
    z = jnp.concatenate([hs, hf], axis=1)
    wz = _dot(z, wb_ref[...])
    w_ref[...] = wz[:, :64] * wz[:, 64:128] * sht_ref[...].T


def _final_body(pa_ref, pb_ref, pc_ref, t3_ref, wo0_ref, bo0_ref,
                mout_ref, o_ref):
    pa = pa_ref[...]
    pb = pb_ref[...]
    pc = pc_ref[...]
    t3 = t3_ref[...]
    p0 = pa[0] + pa[1] + pc[0]
    p1 = pb[0] + pb[1] + pc[1]
    acc0 = p0[:, :32] + p1[:, :32] + t3[:, :32]
    acc1 = p0[:, 32:] + p1[:, 32:] + t3[:, 32:128]
    o0 = _dot(acc0, wo0_ref[...]) + bo0_ref[...]
    o1 = _dot(acc1, mout_ref[...])
    o_ref[...] = jnp.concatenate([o0, o1], axis=1)


def _make_gather(n, es, ch, si):
    nch = (es // NW) // ch
    epw = es // NW
    mesh = plsc.VectorSubcoreMesh(core_axis_name="c", subcore_axis_name="s")

    @functools.partial(
        pl.kernel,
        out_type=[jax.ShapeDtypeStruct((es, 128), jnp.float32),
                  jax.ShapeDtypeStruct((es, 128), jnp.float32)],
        mesh=mesh,
        scratch_types=[
            pltpu.VMEM((nch, ch), jnp.int32),
            pltpu.VMEM((nch, ch), jnp.int32),
            pltpu.VMEM((ch, 128), jnp.float32),
            pltpu.VMEM((ch, 128), jnp.float32),
            pltpu.SemaphoreType.DMA,
            pltpu.SemaphoreType.DMA,
        ],
    )
    def gather(t1, dsti, srci, g1d, g1s, idxd, idxs, b1, b2, sem1, sem2):
        c = lax.axis_index("c")
        s = lax.axis_index("s")
        wid = s * NC + c
        base = wid * epw
        pltpu.sync_copy(dsti.at[si * NW + wid], idxd)
        pltpu.sync_copy(srci.at[si * NW + wid], idxs)

        def body(j, carry):
            cp1 = pltpu.async_copy(t1.at[idxd.at[j]], b1, sem1)
            cp2 = pltpu.async_copy(t1.at[idxs.at[j]], b2, sem2)
            cp1.wait()
            cp2.wait()
            pltpu.sync_copy(b1, g1d.at[pl.ds(base + j * ch, ch)])
            pltpu.sync_copy(b2, g1s.at[pl.ds(base + j * ch, ch)])
            return carry

        lax.fori_loop(0, nch, body, 0)

    return gather


def _make_scatter(n_pad, es, ch, slabs):
    nch = (es // NW) // ch
    epw = es // NW
    npt = n_pad // NS
    nslab = len(slabs)
    mesh = plsc.VectorSubcoreMesh(core_axis_name="c", subcore_axis_name="s")

    @functools.partial(
        pl.kernel,
        out_type=jax.ShapeDtypeStruct((NC, n_pad, 128), jnp.float32),
        mesh=mesh,
        scratch_types=[
            pltpu.VMEM((nch, ch), jnp.int32),
            pltpu.VMEM((nch, ch), jnp.int32),
            pltpu.VMEM((ch, 64), jnp.float32),
            pltpu.VMEM((ch, 128), jnp.float32),
            pltpu.VMEM((ch, 128), jnp.float32),
            pltpu.VMEM_SHARED((n_pad, 128), jnp.float32),
            pltpu.SemaphoreType.DMA,
        ],
    )
    def scatter(*refs):
        ws = refs[:nslab]
        t3 = refs[nslab]
        dsti = refs[nslab + 1]
        srci = refs[nslab + 2]
        zeros = refs[nslab + 3]
        out = refs[nslab + 4]
        idxd, idxs, bufw, bufx, bufe, shared, sem = refs[nslab + 5:]
        c = lax.axis_index("c")
        s = lax.axis_index("s")
        wid = s * NC + c
        base = wid * epw
        pltpu.sync_copy(zeros.at[pl.ds(s * npt, npt)],
                        shared.at[pl.ds(s * npt, npt)])
        plsc.subcore_barrier()

        for wslab, si in zip(ws, slabs):
            pltpu.sync_copy(dsti.at[si * NW + wid], idxd)
            pltpu.sync_copy(srci.at[si * NW + wid], idxs)

            def body(j, carry):
                cpx = pltpu.async_copy(t3.at[idxs.at[j]], bufx, sem)
                pltpu.sync_copy(wslab.at[pl.ds(base + j * ch, ch)], bufw)
                cpx.wait()

                def edge(p, carry2):
                    wv = [bufw[p, pl.ds(16 * k, 16)] for k in range(4)]
                    sel = [0, 1, 2, 3, 2, 3, 2, 3]
                    for k in range(8):
                        bufe[p, pl.ds(16 * k, 16)] = (
                            bufx[p, pl.ds(16 * k, 16)] * wv[sel[k]])
                    return carry2

                lax.fori_loop(0, ch, edge, 0)
                pltpu.sync_copy(bufe, shared.at[idxd.at[j]], add=True)
                return carry

            lax.fori_loop(0, nch, body, 0)
        plsc.subcore_barrier()
        pltpu.sync_copy(shared.at[pl.ds(s * npt, npt)],
                        out.at[c].at[pl.ds(s * npt, npt)])

    return scatter


def kernel(x, edge_index, edge_sh, edge_attr, W_pre0, b_pre0, W_pre1,
           W_node0, b_node0, W_node1, Wg1, bg1, Wg2, bg2, Wf1, Wf2,
           Wl1, Wl2, Wo0, bo0, Wo1):
    n = x.shape[0]
    e = edge_index.shape[1]
    ch = 80
    n_pad = ((n + 8 * NS - 1) // (8 * NS)) * 8 * NS
    assert e % NW == 0 and (e // NW) % ch == 0

    f32 = jnp.float32
    perm = jnp.asarray(_PERM_NP)
    i3 = jnp.eye(3, dtype=f32)
    s32 = 1.0 / math.sqrt(32.0)

    m_pre1 = (W_pre1[:, None, None, :] * i3[None, :, :, None]
              * s32).reshape(96, 96)
    m_node1 = (i3[:, None, :, None] * W_node1[None, :, None, :]
               * s32).reshape(96, 96)
    wp = (Wl1[:32] + Wl1[32:64]) / math.sqrt(96.0)
    wc = Wl1[64:96] / math.sqrt(96.0)
    wl2 = Wl2 / math.sqrt(32.0)
    wf1 = Wf1 / math.sqrt(16.0)
    wf2 = Wf2 / math.sqrt(32.0)
    m_out = (i3[:, None, None, :] * Wo1[None, :, :, None]
             * s32).reshape(96, 96)
    wo0 = Wo0 * s32
    bpre0 = b_pre0.reshape(1, 32)
    bnode0 = b_node0.reshape(1, 32)
    bg1r = bg1.reshape(1, 64)
    bg2r = bg2.reshape(1, 64)
    bo0r = bo0.reshape(1, 32)

    wa = jnp.zeros((48, 64), f32)
    wa = wa.at[:32, :32].set(wc).at[32:48, 32:64].set(wf1)
    col_scale = jnp.concatenate([jnp.full((32,), _C0, f32),
                                 jnp.full((32,), _C1, f32)])
    wb = jnp.zeros((64, 128), f32)
    wb = wb.at[:32, :64].set(wl2 * col_scale[None, :]).at[32:64, 64:128].set(wf2)

    bn = 2000
    gn = n // bn
    be = 6400

    def full(shape):
        return pl.BlockSpec(shape, lambda i: (0,) * len(shape))

    t1 = pl.pallas_call(
        _node_a_body,
        grid=(gn,),
        in_specs=[
            pl.BlockSpec((bn, 128), lambda i: (i, 0)),
            full((96, 96)), full((32, 32)), full((1, 32)), full((96, 96)),
            full((32, 32)),
        ],
        out_specs=pl.BlockSpec((bn, 128), lambda i: (i, 0)),
        out_shape=jax.ShapeDtypeStruct((n, 128), f32),
        compiler_params=pltpu.CompilerParams(
            dimension_semantics=("arbitrary",)),
    )(x, perm, W_pre0 * s32, bpre0, m_pre1, wp)
    t3 = pl.pallas_call(
        _node_b_body,
        grid=(gn,),
        in_specs=[
            pl.BlockSpec((bn, 128), lambda i: (i, 0)),
            full((96, 96)), full((64, 64)), full((1, 64)), full((64, 64)),
            full((1, 64)), full((32, 32)), full((1, 32)), full((96, 96)),
        ],
        out_specs=pl.BlockSpec((bn, 128), lambda i: (i, 0)),
        out_shape=jax.ShapeDtypeStruct((n, 128), f32),
        compiler_params=pltpu.CompilerParams(
            dimension_semantics=("arbitrary",)),
    )(x, perm, Wg1, bg1r, Wg2, bg2r, W_node0 * s32, bnode0, m_node1)

    nslab = 5
    es = e // nslab
    epw = es // NW
    nch = epw // ch
    assert es % NW == 0 and epw % ch == 0 and es % be == 0
    ges = es // be
    dst3 = edge_index[0].reshape(nslab * NW, nch, ch)
    src3 = edge_index[1].reshape(nslab * NW, nch, ch)
    attr_t = edge_attr.T
    sh_t = edge_sh.reshape(1, e)
    w_slabs = []
    for si in range(nslab):
        g1d, g1s = _make_gather(n, es, ch, si)(t1, dst3, src3)
        base_blk = si * ges
        ws = pl.pallas_call(
            _edge_body,
            grid=(ges,),
            in_specs=[
                pl.BlockSpec((be, 128), lambda i: (i, 0)),
                pl.BlockSpec((be, 128), lambda i: (i, 0)),
                pl.BlockSpec((16, be), lambda i, b=base_blk: (0, b + i)),
                pl.BlockSpec((1, be), lambda i, b=base_blk: (0, b + i)),
                full((48, 64)), full((64, 128)),
            ],
            out_specs=pl.BlockSpec((be, 64), lambda i: (i, 0)),
            out_shape=jax.ShapeDtypeStruct((es, 64), f32),
            compiler_params=pltpu.CompilerParams(
                dimension_semantics=("arbitrary",)),
        )(g1d, g1s, attr_t, sh_t, wa, wb)
        w_slabs.append(ws)

    zeros = jnp.zeros((n_pad, 128), f32)
    pacc_a = _make_scatter(n_pad, es, ch, (0, 1))(
        w_slabs[0], w_slabs[1], t3, dst3, src3, zeros)
    pacc_b = _make_scatter(n_pad, es, ch, (2, 3))(
        w_slabs[2], w_slabs[3], t3, dst3, src3, zeros)
    pacc_c = _make_scatter(n_pad, es, ch, (4,))(
        w_slabs[4], t3, dst3, src3, zeros)

    out = pl.pallas_call(
        _final_body,
        grid=(gn,),
        in_specs=[
            pl.BlockSpec((NC, bn, 128), lambda i: (0, i, 0)),
            pl.BlockSpec((NC, bn, 128), lambda i: (0, i, 0)),
            pl.BlockSpec((NC, bn, 128), lambda i: (0, i, 0)),
            pl.BlockSpec((bn, 128), lambda i: (i, 0)),
            full((32, 32)), full((1, 32)), full((96, 96)),
        ],
        out_specs=pl.BlockSpec((bn, 128), lambda i: (i, 0)),
        out_shape=jax.ShapeDtypeStruct((n, 128), f32),
        compiler_params=pltpu.CompilerParams(
            dimension_semantics=("arbitrary",)),
    )(pacc_a, pacc_b, pacc_c, t3, wo0, bo0r, m_out)

    return out
